# baseline (speedup 1.0000x reference)
import jax
import jax.numpy as jnp
from jax import lax
from jax.experimental import pallas as pl
from jax.experimental.pallas import tpu as pltpu
from jax.experimental.pallas import tpu_sc as plsc

_N = 10000
_E = 320000
_F = 128
_CH = 128
_NC, _NS = 2, 16
_NW = _NC * _NS
_CPW = 80
_PAD_ROWS = _NW * _CPW
_NSENT = 16
_NPAD = _N + _NSENT
_RMAIN = 624
_R = 2504

_sc_mesh = plsc.VectorSubcoreMesh(core_axis_name="c", subcore_axis_name="s")


def _copy_rows(src, dst, s):
    base = s * _RMAIN
    pltpu.sync_copy(src.at[pl.ds(base, _RMAIN)], dst.at[pl.ds(base, _RMAIN)])

    @pl.when(s == 0)
    def _():
        rb = _NS * _RMAIN
        pltpu.sync_copy(src.at[pl.ds(rb, 16)], dst.at[pl.ds(rb, 16)])


def _zero_rows(zsrc, dst, s):
    base = s * _RMAIN
    pltpu.sync_copy(zsrc.at[pl.ds(0, _RMAIN)], dst.at[pl.ds(base, _RMAIN)])

    @pl.when(s == 0)
    def _():
        rb = _NS * _RMAIN
        pltpu.sync_copy(zsrc.at[pl.ds(0, 16)], dst.at[pl.ds(rb, 16)])


def _hist_body(dst_hbm, ones_hbm, zeros_hbm, out_hbm, hist_sh, didx_v, ones_v):
    c = lax.axis_index("c")
    s = lax.axis_index("s")
    wid = c * _NS + s
    _zero_rows(zeros_hbm, hist_sh, s)
    pltpu.sync_copy(ones_hbm, ones_v)
    pltpu.sync_copy(dst_hbm.at[pl.ds(wid * _CPW, _CPW)], didx_v)
    plsc.subcore_barrier()

    def step(j, carry):
        pltpu.sync_copy(ones_v, hist_sh.at[didx_v.at[j]], add=True)
        return carry

    lax.fori_loop(0, _CPW, step, 0)
    plsc.subcore_barrier()
    _copy_rows(hist_sh, out_hbm.at[c], s)


_hist_call = pl.kernel(
    _hist_body,
    out_type=jax.ShapeDtypeStruct((_NC, _NPAD, _F), jnp.float32),
    mesh=_sc_mesh,
    scratch_types=[
        pltpu.VMEM_SHARED((_NPAD, _F), jnp.float32),
        pltpu.VMEM((_CPW, _CH), jnp.int32),
        pltpu.VMEM((_CH, _F), jnp.float32),
    ],
)


def _prop_body(y_hbm, zeros_hbm, src_hbm, dst_hbm, out_hbm,
               acc_sh, sidx_v, didx_v, rows0, rows1, sem0, sem1):
    c = lax.axis_index("c")
    s = lax.axis_index("s")
    wid = c * _NS + s

    @pl.when(c == 0)
    def _():
        _copy_rows(y_hbm, acc_sh, s)

    @pl.when(c == 1)
    def _():
        _zero_rows(zeros_hbm, acc_sh, s)

    plsc.subcore_barrier()

    _HC = _CPW // 2
    for h in range(2):
        hb = wid * _CPW + h * _HC
        pltpu.sync_copy(src_hbm.at[pl.ds(hb, _HC)], sidx_v)
        pltpu.sync_copy(dst_hbm.at[pl.ds(hb, _HC)], didx_v)
        pltpu.async_copy(y_hbm.at[sidx_v.at[0]], rows0, sem0)

        def step(jj, carry):
            j0 = 2 * jj
            pltpu.async_copy(y_hbm.at[sidx_v.at[j0 + 1]], rows1, sem1)
            pltpu.make_async_copy(y_hbm.at[sidx_v.at[j0]], rows0, sem0).wait()
            pltpu.sync_copy(rows0, acc_sh.at[didx_v.at[j0]], add=True)

            @pl.when(jj < _HC // 2 - 1)
            def _():
                pltpu.async_copy(y_hbm.at[sidx_v.at[j0 + 2]], rows0, sem0)

            pltpu.make_async_copy(y_hbm.at[sidx_v.at[j0 + 1]], rows1, sem1).wait()
            pltpu.sync_copy(rows1, acc_sh.at[didx_v.at[j0 + 1]], add=True)
            return carry

        lax.fori_loop(0, _HC // 2, step, 0)
    plsc.subcore_barrier()
    _copy_rows(acc_sh, out_hbm.at[c], s)


_prop_call = pl.kernel(
    _prop_body,
    out_type=jax.ShapeDtypeStruct((_NC, _NPAD, _F), jnp.float32),
    mesh=_sc_mesh,
    scratch_types=[
        pltpu.VMEM_SHARED((_NPAD, _F), jnp.float32),
        pltpu.VMEM((_CPW // 2, _CH), jnp.int32),
        pltpu.VMEM((_CPW // 2, _CH), jnp.int32),
        pltpu.VMEM((_CH, _F), jnp.float32),
        pltpu.VMEM((_CH, _F), jnp.float32),
        pltpu.SemaphoreType.DMA,
        pltpu.SemaphoreType.DMA,
    ],
)


def _dinv_block(hist_ref):
    cnt = hist_ref[0, :, 0:1] + hist_ref[1, :, 0:1]
    return lax.rsqrt(cnt + 1.0)


def _b1_body(x_ref, w_ref, b_ref, h_ref):
    h = jnp.dot(x_ref[...], w_ref[...], preferred_element_type=jnp.float32)
    h_ref[...] = h + b_ref[...]


_stage_b1 = pl.pallas_call(
    _b1_body,
    grid=(_NPAD // _R,),
    in_specs=[
        pl.BlockSpec((_R, 128), lambda i: (i, 0)),
        pl.BlockSpec((128, 128), lambda i: (0, 0)),
        pl.BlockSpec((1, 128), lambda i: (0, 0)),
    ],
    out_specs=pl.BlockSpec((_R, 128), lambda i: (i, 0)),
    out_shape=jax.ShapeDtypeStruct((_NPAD, 128), jnp.float32),
)


def _b2_body(hist_ref, h_ref, y_ref, dv_ref):
    dinv = _dinv_block(hist_ref)
    y_ref[...] = dinv * h_ref[...]
    dv_ref[...] = jnp.broadcast_to(dinv, (_R, 8))


_stage_b2 = pl.pallas_call(
    _b2_body,
    grid=(_NPAD // _R,),
    in_specs=[
        pl.BlockSpec((_NC, _R, _F), lambda i: (0, i, 0)),
        pl.BlockSpec((_R, 128), lambda i: (i, 0)),
    ],
    out_specs=[pl.BlockSpec((_R, 128), lambda i: (i, 0)),
               pl.BlockSpec((_R, 8), lambda i: (i, 0))],
    out_shape=[jax.ShapeDtypeStruct((_NPAD, 128), jnp.float32),
               jax.ShapeDtypeStruct((_NPAD, 8), jnp.float32)],
)


def _d_body(dv_ref, a_ref, g_ref, bt_ref, wmu_ref, bmu_ref, y_ref):
    dinv = dv_ref[:, 0:1]
    h = dinv * (a_ref[0] + a_ref[1])
    h = h * (g_ref[...] * (1.0 / jnp.sqrt(1.0 + 1e-5))) + bt_ref[...]
    h = jnp.where(h >= 0, h, 0.01 * h)
    p = jnp.dot(h, wmu_ref[...], preferred_element_type=jnp.float32) + bmu_ref[...]
    ss = jnp.sum(p * p, axis=1, keepdims=True)
    nrm = jnp.maximum(jnp.sqrt(ss), 1e-12)
    q = p * (1.8 / nrm)
    y_ref[...] = dinv * jnp.concatenate([p, q], axis=1)


_stage_d = pl.pallas_call(
    _d_body,
    grid=(_NPAD // _R,),
    in_specs=[
        pl.BlockSpec((_R, 8), lambda i: (i, 0)),
        pl.BlockSpec((_NC, _R, 128), lambda i: (0, i, 0)),
        pl.BlockSpec((1, 128), lambda i: (0, 0)),
        pl.BlockSpec((1, 128), lambda i: (0, 0)),
        pl.BlockSpec((128, 64), lambda i: (0, 0)),
        pl.BlockSpec((1, 64), lambda i: (0, 0)),
    ],
    out_specs=pl.BlockSpec((_R, 128), lambda i: (i, 0)),
    out_shape=jax.ShapeDtypeStruct((_NPAD, 128), jnp.float32),
)


def _f_body(dv_ref, a_ref, mu_ref, ls_ref):
    z = dv_ref[:, 0:1] * (a_ref[0] + a_ref[1])
    mu_ref[...] = z[:, 0:64]
    ls_ref[...] = z[:, 64:128]


_stage_f = pl.pallas_call(
    _f_body,
    grid=(_NPAD // _R,),
    in_specs=[
        pl.BlockSpec((_R, 8), lambda i: (i, 0)),
        pl.BlockSpec((_NC, _R, 128), lambda i: (0, i, 0)),
    ],
    out_specs=[pl.BlockSpec((_R, 64), lambda i: (i, 0)),
               pl.BlockSpec((_R, 64), lambda i: (i, 0))],
    out_shape=[jax.ShapeDtypeStruct((_N, 64), jnp.float32),
               jax.ShapeDtypeStruct((_N, 64), jnp.float32)],
)


def kernel(x, edge_index, W1, b1, bn_gamma, bn_beta, Wmu, bmu):
    pad = _PAD_ROWS * _CH - _E
    sent = (_N + (jnp.arange(pad, dtype=edge_index.dtype) % _NSENT))
    src2d = jnp.concatenate([edge_index[0], sent]).reshape(_PAD_ROWS, _CH)
    dst2d = jnp.concatenate([edge_index[1], sent]).reshape(_PAD_ROWS, _CH)
    ones128 = jnp.ones((_CH, _F), jnp.float32)
    zeros624 = jnp.zeros((_RMAIN, _F), jnp.float32)

    hist = _hist_call(dst2d, ones128, zeros624)
    h1 = _stage_b1(x, W1.T, b1.reshape(1, -1))
    y1, dinv8 = _stage_b2(hist, h1)
    acc1 = _prop_call(y1, zeros624, src2d, dst2d)
    y2 = _stage_d(dinv8, acc1, bn_gamma.reshape(1, -1), bn_beta.reshape(1, -1),
                  Wmu.T, bmu.reshape(1, -1))
    acc2 = _prop_call(y2, zeros624, src2d, dst2d)
    mu, logstd = _stage_f(dinv8, acc2)
    return (mu, logstd, mu)


# speedup vs baseline: 1.0308x; 1.0308x over previous
import jax
import jax.numpy as jnp
from jax import lax
from jax.experimental import pallas as pl
from jax.experimental.pallas import tpu as pltpu
from jax.experimental.pallas import tpu_sc as plsc

_N = 10000
_E = 320000
_F = 128
_CH = 128
_NC, _NS = 2, 16
_NW = _NC * _NS
_CPW = 80
_PAD_ROWS = _NW * _CPW
_NSENT = 16
_NPAD = _N + _NSENT
_RMAIN = 624
_R = 2504

_sc_mesh = plsc.VectorSubcoreMesh(core_axis_name="c", subcore_axis_name="s")


def _copy_rows(src, dst, s):
    base = s * _RMAIN
    pltpu.sync_copy(src.at[pl.ds(base, _RMAIN)], dst.at[pl.ds(base, _RMAIN)])

    @pl.when(s == 0)
    def _():
        rb = _NS * _RMAIN
        pltpu.sync_copy(src.at[pl.ds(rb, 16)], dst.at[pl.ds(rb, 16)])


def _zero_rows(zsrc, dst, s):
    base = s * _RMAIN
    pltpu.sync_copy(zsrc.at[pl.ds(0, _RMAIN)], dst.at[pl.ds(base, _RMAIN)])

    @pl.when(s == 0)
    def _():
        rb = _NS * _RMAIN
        pltpu.sync_copy(zsrc.at[pl.ds(0, 16)], dst.at[pl.ds(rb, 16)])


def _hist_body(eidx_hbm, ones_hbm, zeros_hbm, out_hbm, hist_sh, didx_v, ones_v):
    c = lax.axis_index("c")
    s = lax.axis_index("s")
    wid = c * _NS + s
    _zero_rows(zeros_hbm, hist_sh, s)
    pltpu.sync_copy(ones_hbm, ones_v)
    pltpu.sync_copy(eidx_hbm.at[1, pl.ds(wid * _CPW, _CPW)], didx_v)
    plsc.subcore_barrier()

    def step(j, carry):
        pltpu.sync_copy(ones_v, hist_sh.at[didx_v.at[j]], add=True)
        return carry

    lax.fori_loop(0, _CPW, step, 0)
    plsc.subcore_barrier()
    _copy_rows(hist_sh, out_hbm.at[c], s)


_hist_call = pl.kernel(
    _hist_body,
    out_type=jax.ShapeDtypeStruct((_NC, _NPAD, _F), jnp.float32),
    mesh=_sc_mesh,
    scratch_types=[
        pltpu.VMEM_SHARED((_NPAD, _F), jnp.float32),
        pltpu.VMEM((_CPW, _CH), jnp.int32),
        pltpu.VMEM((_CH, _F), jnp.float32),
    ],
)


def _prop_body(y_hbm, zeros_hbm, eidx_hbm, out_hbm,
               acc_sh, sidx_v, didx_v, rows0, rows1, sem0, sem1):
    c = lax.axis_index("c")
    s = lax.axis_index("s")
    wid = c * _NS + s

    @pl.when(c == 0)
    def _():
        _copy_rows(y_hbm, acc_sh, s)

    @pl.when(c == 1)
    def _():
        _zero_rows(zeros_hbm, acc_sh, s)

    plsc.subcore_barrier()

    _HC = _CPW // 2
    for h in range(2):
        hb = wid * _CPW + h * _HC
        pltpu.sync_copy(eidx_hbm.at[0, pl.ds(hb, _HC)], sidx_v)
        pltpu.sync_copy(eidx_hbm.at[1, pl.ds(hb, _HC)], didx_v)
        pltpu.async_copy(y_hbm.at[sidx_v.at[0]], rows0, sem0)

        def step(jj, carry):
            j0 = 2 * jj
            pltpu.async_copy(y_hbm.at[sidx_v.at[j0 + 1]], rows1, sem1)
            pltpu.make_async_copy(y_hbm.at[sidx_v.at[j0]], rows0, sem0).wait()
            pltpu.sync_copy(rows0, acc_sh.at[didx_v.at[j0]], add=True)

            @pl.when(jj < _HC // 2 - 1)
            def _():
                pltpu.async_copy(y_hbm.at[sidx_v.at[j0 + 2]], rows0, sem0)

            pltpu.make_async_copy(y_hbm.at[sidx_v.at[j0 + 1]], rows1, sem1).wait()
            pltpu.sync_copy(rows1, acc_sh.at[didx_v.at[j0 + 1]], add=True)
            return carry

        lax.fori_loop(0, _HC // 2, step, 0)
    plsc.subcore_barrier()
    _copy_rows(acc_sh, out_hbm.at[c], s)


_prop_call = pl.kernel(
    _prop_body,
    out_type=jax.ShapeDtypeStruct((_NC, _NPAD, _F), jnp.float32),
    mesh=_sc_mesh,
    scratch_types=[
        pltpu.VMEM_SHARED((_NPAD, _F), jnp.float32),
        pltpu.VMEM((_CPW // 2, _CH), jnp.int32),
        pltpu.VMEM((_CPW // 2, _CH), jnp.int32),
        pltpu.VMEM((_CH, _F), jnp.float32),
        pltpu.VMEM((_CH, _F), jnp.float32),
        pltpu.SemaphoreType.DMA,
        pltpu.SemaphoreType.DMA,
    ],
)


_EB = 512


def _e_body(e_ref, o_ref):
    i = pl.program_id(1)
    row = lax.broadcasted_iota(jnp.int32, (_EB, _CH), 0) + i * _EB
    lane = lax.broadcasted_iota(jnp.int32, (_EB, _CH), 1)
    sent = _N + (lane % _NSENT)
    o_ref[0] = jnp.where(row < _E // _CH, e_ref[0], sent)


_stage_e = pl.pallas_call(
    _e_body,
    grid=(2, _PAD_ROWS // _EB),
    in_specs=[pl.BlockSpec((1, _EB, _CH), lambda j, i: (j, i, 0))],
    out_specs=pl.BlockSpec((1, _EB, _CH), lambda j, i: (j, i, 0)),
    out_shape=jax.ShapeDtypeStruct((2, _PAD_ROWS, _CH), jnp.int32),
)


def _dinv_block(hist_ref):
    cnt = hist_ref[0, :, 0:1] + hist_ref[1, :, 0:1]
    return lax.rsqrt(cnt + 1.0)


def _b1_body(x_ref, w_ref, b_ref, h_ref):
    h = jnp.dot(x_ref[...], w_ref[...], preferred_element_type=jnp.float32)
    h_ref[...] = h + b_ref[...]


_stage_b1 = pl.pallas_call(
    _b1_body,
    grid=(_NPAD // _R,),
    in_specs=[
        pl.BlockSpec((_R, 128), lambda i: (i, 0)),
        pl.BlockSpec((128, 128), lambda i: (0, 0)),
        pl.BlockSpec((1, 128), lambda i: (0, 0)),
    ],
    out_specs=pl.BlockSpec((_R, 128), lambda i: (i, 0)),
    out_shape=jax.ShapeDtypeStruct((_NPAD, 128), jnp.float32),
)


def _b2_body(hist_ref, h_ref, y_ref, dv_ref):
    dinv = _dinv_block(hist_ref)
    y_ref[...] = dinv * h_ref[...]
    dv_ref[...] = jnp.broadcast_to(dinv, (_R, 8))


_stage_b2 = pl.pallas_call(
    _b2_body,
    grid=(_NPAD // _R,),
    in_specs=[
        pl.BlockSpec((_NC, _R, _F), lambda i: (0, i, 0)),
        pl.BlockSpec((_R, 128), lambda i: (i, 0)),
    ],
    out_specs=[pl.BlockSpec((_R, 128), lambda i: (i, 0)),
               pl.BlockSpec((_R, 8), lambda i: (i, 0))],
    out_shape=[jax.ShapeDtypeStruct((_NPAD, 128), jnp.float32),
               jax.ShapeDtypeStruct((_NPAD, 8), jnp.float32)],
)


def _d_body(dv_ref, a_ref, g_ref, bt_ref, wmu_ref, bmu_ref, y_ref):
    dinv = dv_ref[:, 0:1]
    h = dinv * (a_ref[0] + a_ref[1])
    h = h * (g_ref[...] * (1.0 / jnp.sqrt(1.0 + 1e-5))) + bt_ref[...]
    h = jnp.where(h >= 0, h, 0.01 * h)
    p = jnp.dot(h, wmu_ref[...], preferred_element_type=jnp.float32) + bmu_ref[...]
    ss = jnp.sum(p * p, axis=1, keepdims=True)
    nrm = jnp.maximum(jnp.sqrt(ss), 1e-12)
    q = p * (1.8 / nrm)
    y_ref[...] = dinv * jnp.concatenate([p, q], axis=1)


_stage_d = pl.pallas_call(
    _d_body,
    grid=(_NPAD // _R,),
    in_specs=[
        pl.BlockSpec((_R, 8), lambda i: (i, 0)),
        pl.BlockSpec((_NC, _R, 128), lambda i: (0, i, 0)),
        pl.BlockSpec((1, 128), lambda i: (0, 0)),
        pl.BlockSpec((1, 128), lambda i: (0, 0)),
        pl.BlockSpec((128, 64), lambda i: (0, 0)),
        pl.BlockSpec((1, 64), lambda i: (0, 0)),
    ],
    out_specs=pl.BlockSpec((_R, 128), lambda i: (i, 0)),
    out_shape=jax.ShapeDtypeStruct((_NPAD, 128), jnp.float32),
)


def _f_body(dv_ref, a_ref, mu_ref, ls_ref):
    z = dv_ref[:, 0:1] * (a_ref[0] + a_ref[1])
    mu_ref[...] = z[:, 0:64]
    ls_ref[...] = z[:, 64:128]


_stage_f = pl.pallas_call(
    _f_body,
    grid=(_NPAD // _R,),
    in_specs=[
        pl.BlockSpec((_R, 8), lambda i: (i, 0)),
        pl.BlockSpec((_NC, _R, 128), lambda i: (0, i, 0)),
    ],
    out_specs=[pl.BlockSpec((_R, 64), lambda i: (i, 0)),
               pl.BlockSpec((_R, 64), lambda i: (i, 0))],
    out_shape=[jax.ShapeDtypeStruct((_N, 64), jnp.float32),
               jax.ShapeDtypeStruct((_N, 64), jnp.float32)],
)


def kernel(x, edge_index, W1, b1, bn_gamma, bn_beta, Wmu, bmu):
    ones128 = jnp.ones((_CH, _F), jnp.float32)
    zeros624 = jnp.zeros((_RMAIN, _F), jnp.float32)
    eidx = _stage_e(edge_index.reshape(2, _E // _CH, _CH))

    hist = _hist_call(eidx, ones128, zeros624)
    h1 = _stage_b1(x, W1.T, b1.reshape(1, -1))
    y1, dinv8 = _stage_b2(hist, h1)
    acc1 = _prop_call(y1, zeros624, eidx)
    y2 = _stage_d(dinv8, acc1, bn_gamma.reshape(1, -1), bn_beta.reshape(1, -1),
                  Wmu.T, bmu.reshape(1, -1))
    acc2 = _prop_call(y2, zeros624, eidx)
    mu, logstd = _stage_f(dinv8, acc2)
    return (mu, logstd, mu)


# speedup vs baseline: 1.0344x; 1.0034x over previous
import jax
import jax.numpy as jnp
from jax import lax
from jax.experimental import pallas as pl
from jax.experimental.pallas import tpu as pltpu
from jax.experimental.pallas import tpu_sc as plsc

_N = 10000
_E = 320000
_F = 128
_CH = 128
_NC, _NS = 2, 16
_NW = _NC * _NS
_CPW = 80
_PAD_ROWS = _NW * _CPW
_NSENT = 16
_NPAD = _N + _NSENT
_RMAIN = 624
_R = 2504

_sc_mesh = plsc.VectorSubcoreMesh(core_axis_name="c", subcore_axis_name="s")


def _copy_rows(src, dst, s):
    base = s * _RMAIN
    pltpu.sync_copy(src.at[pl.ds(base, _RMAIN)], dst.at[pl.ds(base, _RMAIN)])

    @pl.when(s == 0)
    def _():
        rb = _NS * _RMAIN
        pltpu.sync_copy(src.at[pl.ds(rb, 16)], dst.at[pl.ds(rb, 16)])


def _zero_rows(zsrc, dst, s):
    base = s * _RMAIN
    pltpu.sync_copy(zsrc.at[pl.ds(0, _RMAIN)], dst.at[pl.ds(base, _RMAIN)])

    @pl.when(s == 0)
    def _():
        rb = _NS * _RMAIN
        pltpu.sync_copy(zsrc.at[pl.ds(0, 16)], dst.at[pl.ds(rb, 16)])


def _hist_body(eidx_hbm, ones_hbm, zeros_hbm, out_hbm, hist_sh, didx_v, ones_v,
               sem0):
    c = lax.axis_index("c")
    s = lax.axis_index("s")
    wid = c * _NS + s
    _zero_rows(zeros_hbm, hist_sh, s)
    pltpu.sync_copy(ones_hbm, ones_v)
    pltpu.sync_copy(eidx_hbm.at[1, pl.ds(wid * _CPW, _CPW)], didx_v)
    plsc.subcore_barrier()

    def grp(g, carry):
        def fire(k, carry2):
            pltpu.async_copy(ones_v, hist_sh.at[didx_v.at[g * 8 + k]],
                             sem0, add=True)
            return carry2

        lax.fori_loop(0, 8, fire, 0)

        def drain(k, carry2):
            pltpu.make_async_copy(ones_v, hist_sh.at[didx_v.at[g * 8 + k]],
                                  sem0).wait()
            return carry2

        lax.fori_loop(0, 8, drain, 0)
        return carry

    lax.fori_loop(0, _CPW // 8, grp, 0)
    plsc.subcore_barrier()
    _copy_rows(hist_sh, out_hbm.at[c], s)


_hist_call = pl.kernel(
    _hist_body,
    out_type=jax.ShapeDtypeStruct((_NC, _NPAD, _F), jnp.float32),
    mesh=_sc_mesh,
    scratch_types=[
        pltpu.VMEM_SHARED((_NPAD, _F), jnp.float32),
        pltpu.VMEM((_CPW, _CH), jnp.int32),
        pltpu.VMEM((_CH, _F), jnp.float32),
        pltpu.SemaphoreType.DMA,
    ],
)


def _prop_body(y_hbm, zeros_hbm, eidx_hbm, out_hbm,
               acc_sh, sidx_v, didx_v, rows0, rows1, sem0, sem1):
    c = lax.axis_index("c")
    s = lax.axis_index("s")
    wid = c * _NS + s

    _HC = _CPW // 2
    pltpu.sync_copy(eidx_hbm.at[0, pl.ds(wid * _CPW, _HC)], sidx_v)
    pltpu.sync_copy(eidx_hbm.at[1, pl.ds(wid * _CPW, _HC)], didx_v)

    @pl.when(c == 0)
    def _():
        _copy_rows(y_hbm, acc_sh, s)

    @pl.when(c == 1)
    def _():
        _zero_rows(zeros_hbm, acc_sh, s)

    plsc.subcore_barrier()

    for h in range(2):
        hb = wid * _CPW + h * _HC

        if h > 0:
            pltpu.sync_copy(eidx_hbm.at[0, pl.ds(hb, _HC)], sidx_v)
            pltpu.sync_copy(eidx_hbm.at[1, pl.ds(hb, _HC)], didx_v)

        pltpu.async_copy(y_hbm.at[sidx_v.at[0]], rows0, sem0)

        def step(jj, carry):
            j0 = 2 * jj
            pltpu.async_copy(y_hbm.at[sidx_v.at[j0 + 1]], rows1, sem1)
            pltpu.make_async_copy(y_hbm.at[sidx_v.at[j0]], rows0, sem0).wait()
            pltpu.sync_copy(rows0, acc_sh.at[didx_v.at[j0]], add=True)

            @pl.when(jj < _HC // 2 - 1)
            def _():
                pltpu.async_copy(y_hbm.at[sidx_v.at[j0 + 2]], rows0, sem0)

            pltpu.make_async_copy(y_hbm.at[sidx_v.at[j0 + 1]], rows1, sem1).wait()
            pltpu.sync_copy(rows1, acc_sh.at[didx_v.at[j0 + 1]], add=True)
            return carry

        lax.fori_loop(0, _HC // 2, step, 0)
    plsc.subcore_barrier()
    _copy_rows(acc_sh, out_hbm.at[c], s)


_prop_call = pl.kernel(
    _prop_body,
    out_type=jax.ShapeDtypeStruct((_NC, _NPAD, _F), jnp.float32),
    mesh=_sc_mesh,
    scratch_types=[
        pltpu.VMEM_SHARED((_NPAD, _F), jnp.float32),
        pltpu.VMEM((_CPW // 2, _CH), jnp.int32),
        pltpu.VMEM((_CPW // 2, _CH), jnp.int32),
        pltpu.VMEM((_CH, _F), jnp.float32),
        pltpu.VMEM((_CH, _F), jnp.float32),
        pltpu.SemaphoreType.DMA,
        pltpu.SemaphoreType.DMA,
    ],
)


_EB = 512


def _e_body(e_ref, o_ref):
    i = pl.program_id(1)
    row = lax.broadcasted_iota(jnp.int32, (_EB, _CH), 0) + i * _EB
    lane = lax.broadcasted_iota(jnp.int32, (_EB, _CH), 1)
    sent = _N + (lane % _NSENT)
    o_ref[0] = jnp.where(row < _E // _CH, e_ref[0], sent)


_stage_e = pl.pallas_call(
    _e_body,
    grid=(2, _PAD_ROWS // _EB),
    in_specs=[pl.BlockSpec((1, _EB, _CH), lambda j, i: (j, i, 0))],
    out_specs=pl.BlockSpec((1, _EB, _CH), lambda j, i: (j, i, 0)),
    out_shape=jax.ShapeDtypeStruct((2, _PAD_ROWS, _CH), jnp.int32),
)


def _dinv_block(hist_ref):
    cnt = hist_ref[0, :, 0:1] + hist_ref[1, :, 0:1]
    return lax.rsqrt(cnt + 1.0)


def _b1_body(x_ref, w_ref, b_ref, h_ref):
    h = jnp.dot(x_ref[...], w_ref[...], preferred_element_type=jnp.float32)
    h_ref[...] = h + b_ref[...]


_stage_b1 = pl.pallas_call(
    _b1_body,
    grid=(_NPAD // _R,),
    in_specs=[
        pl.BlockSpec((_R, 128), lambda i: (i, 0)),
        pl.BlockSpec((128, 128), lambda i: (0, 0)),
        pl.BlockSpec((1, 128), lambda i: (0, 0)),
    ],
    out_specs=pl.BlockSpec((_R, 128), lambda i: (i, 0)),
    out_shape=jax.ShapeDtypeStruct((_NPAD, 128), jnp.float32),
)


def _b2_body(hist_ref, h_ref, y_ref, dv_ref):
    dinv = _dinv_block(hist_ref)
    y_ref[...] = dinv * h_ref[...]
    dv_ref[...] = jnp.broadcast_to(dinv, (_R, 8))


_stage_b2 = pl.pallas_call(
    _b2_body,
    grid=(_NPAD // _R,),
    in_specs=[
        pl.BlockSpec((_NC, _R, _F), lambda i: (0, i, 0)),
        pl.BlockSpec((_R, 128), lambda i: (i, 0)),
    ],
    out_specs=[pl.BlockSpec((_R, 128), lambda i: (i, 0)),
               pl.BlockSpec((_R, 8), lambda i: (i, 0))],
    out_shape=[jax.ShapeDtypeStruct((_NPAD, 128), jnp.float32),
               jax.ShapeDtypeStruct((_NPAD, 8), jnp.float32)],
)


def _d_body(dv_ref, a_ref, g_ref, bt_ref, wmu_ref, bmu_ref, y_ref):
    dinv = dv_ref[:, 0:1]
    h = dinv * (a_ref[0] + a_ref[1])
    h = h * (g_ref[...] * (1.0 / jnp.sqrt(1.0 + 1e-5))) + bt_ref[...]
    h = jnp.where(h >= 0, h, 0.01 * h)
    p = jnp.dot(h, wmu_ref[...], preferred_element_type=jnp.float32) + bmu_ref[...]
    ss = jnp.sum(p * p, axis=1, keepdims=True)
    nrm = jnp.maximum(jnp.sqrt(ss), 1e-12)
    q = p * (1.8 / nrm)
    y_ref[...] = dinv * jnp.concatenate([p, q], axis=1)


_stage_d = pl.pallas_call(
    _d_body,
    grid=(_NPAD // _R,),
    in_specs=[
        pl.BlockSpec((_R, 8), lambda i: (i, 0)),
        pl.BlockSpec((_NC, _R, 128), lambda i: (0, i, 0)),
        pl.BlockSpec((1, 128), lambda i: (0, 0)),
        pl.BlockSpec((1, 128), lambda i: (0, 0)),
        pl.BlockSpec((128, 64), lambda i: (0, 0)),
        pl.BlockSpec((1, 64), lambda i: (0, 0)),
    ],
    out_specs=pl.BlockSpec((_R, 128), lambda i: (i, 0)),
    out_shape=jax.ShapeDtypeStruct((_NPAD, 128), jnp.float32),
)


def _f_body(dv_ref, a_ref, mu_ref, ls_ref):
    z = dv_ref[:, 0:1] * (a_ref[0] + a_ref[1])
    mu_ref[...] = z[:, 0:64]
    ls_ref[...] = z[:, 64:128]


_stage_f = pl.pallas_call(
    _f_body,
    grid=(_NPAD // _R,),
    in_specs=[
        pl.BlockSpec((_R, 8), lambda i: (i, 0)),
        pl.BlockSpec((_NC, _R, 128), lambda i: (0, i, 0)),
    ],
    out_specs=[pl.BlockSpec((_R, 64), lambda i: (i, 0)),
               pl.BlockSpec((_R, 64), lambda i: (i, 0))],
    out_shape=[jax.ShapeDtypeStruct((_N, 64), jnp.float32),
               jax.ShapeDtypeStruct((_N, 64), jnp.float32)],
)


def kernel(x, edge_index, W1, b1, bn_gamma, bn_beta, Wmu, bmu):
    ones128 = jnp.ones((_CH, _F), jnp.float32)
    zeros624 = jnp.zeros((_RMAIN, _F), jnp.float32)
    eidx = _stage_e(edge_index.reshape(2, _E // _CH, _CH))

    hist = _hist_call(eidx, ones128, zeros624)
    h1 = _stage_b1(x, W1.T, b1.reshape(1, -1))
    y1, dinv8 = _stage_b2(hist, h1)
    acc1 = _prop_call(y1, zeros624, eidx)
    y2 = _stage_d(dinv8, acc1, bn_gamma.reshape(1, -1), bn_beta.reshape(1, -1),
                  Wmu.T, bmu.reshape(1, -1))
    acc2 = _prop_call(y2, zeros624, eidx)
    mu, logstd = _stage_f(dinv8, acc2)
    return (mu, logstd, mu)
